# baseline (speedup 1.0000x reference)
import functools

import jax
import jax.numpy as jnp
from jax import lax
from jax.experimental import pallas as pl
from jax.experimental.pallas import tpu as pltpu
from jax.experimental.pallas import tpu_sc as plsc

N_EMB = 16
N_VXG = 1000000
B = 16384
L = 200
N = B * L

K_PHASES = 5
L_PH = L // K_PHASES
N_PH = L_PH * B

NC = 2
NS = 16
NW = NC * NS
N_PER_W = N_PH // NW
CHUNK = 10240
N_CHUNKS = N_PER_W // CHUNK

B_BLK = 1024
GRID = B // B_BLK


def _sc_gather(phase, ixs_hbm, eff_hbm, out_hbm,
               idx_v0, idx_v1, val_v0, val_v1, eff_sh,
               sem_i0, sem_i1, sem_g, sem_o0, sem_o1):
    sid = lax.axis_index("s")
    wid = sid * NC + lax.axis_index("c")
    base = phase * N_PH + wid * N_PER_W
    obase = wid * N_PER_W

    @pl.when(sid == 0)
    def _():
        pltpu.sync_copy(eff_hbm, eff_sh)

    idx = [idx_v0, idx_v1]
    val = [val_v0, val_v1]
    sem_i = [sem_i0, sem_i1]
    sem_o = [sem_o0, sem_o1]

    pltpu.async_copy(ixs_hbm.at[pl.ds(base, CHUNK)], idx_v0, sem_i0)
    plsc.subcore_barrier()

    for ci in range(N_CHUNKS):
        cur = ci % 2
        nxt = 1 - cur
        if ci + 1 < N_CHUNKS:
            pltpu.async_copy(
                ixs_hbm.at[pl.ds(base + (ci + 1) * CHUNK, CHUNK)],
                idx[nxt], sem_i[nxt])
        pltpu.make_async_copy(
            ixs_hbm.at[pl.ds(base + ci * CHUNK, CHUNK)],
            idx[cur], sem_i[cur]).wait()
        if ci >= 2:
            pltpu.make_async_copy(
                val[cur], out_hbm.at[pl.ds(obase + (ci - 2) * CHUNK, CHUNK)],
                sem_o[cur]).wait()
        pltpu.async_copy(eff_sh.at[idx[cur]], val[cur], sem_g).wait()
        pltpu.async_copy(
            val[cur], out_hbm.at[pl.ds(obase + ci * CHUNK, CHUNK)],
            sem_o[cur])

    for ci in range(max(N_CHUNKS - 2, 0), N_CHUNKS):
        cur = ci % 2
        pltpu.make_async_copy(
            val[cur], out_hbm.at[pl.ds(obase + ci * CHUNK, CHUNK)],
            sem_o[cur]).wait()


def _tc_body(x_ref, g_ref, w_ref, b_ref, *rest):
    o_ref = rest[-1]
    acc = x_ref[:, 0, :] * w_ref[0]
    for k in range(1, N_EMB):
        acc = acc + x_ref[:, k, :] * w_ref[k]
    o_ref[...] = jnp.exp(acc + b_ref[0]) * g_ref[...]


def kernel(variantxgene_embedding, variantxgene_ixs, W, b, variantxgene_effect):
    ixs_raw = (variantxgene_ixs.T.reshape(L // 8, 8, B // 128, 128)
               .transpose(0, 2, 1, 3).reshape(N).astype(jnp.int32))
    x_t = jnp.transpose(variantxgene_embedding, (1, 2, 0))

    mesh = plsc.VectorSubcoreMesh(core_axis_name="c", subcore_axis_name="s")
    acc = None
    for k in range(K_PHASES):
        gather_run = functools.partial(
            pl.kernel,
            mesh=mesh,
            out_type=jax.ShapeDtypeStruct((N_PH,), jnp.float32),
            scratch_types=[
                pltpu.VMEM((CHUNK,), jnp.int32),
                pltpu.VMEM((CHUNK,), jnp.int32),
                pltpu.VMEM((CHUNK,), jnp.float32),
                pltpu.VMEM((CHUNK,), jnp.float32),
                pltpu.VMEM_SHARED((N_VXG,), jnp.float32),
                pltpu.SemaphoreType.DMA,
                pltpu.SemaphoreType.DMA,
                pltpu.SemaphoreType.DMA,
                pltpu.SemaphoreType.DMA,
                pltpu.SemaphoreType.DMA,
            ],
        )(functools.partial(_sc_gather, k))
        gathered = gather_run(ixs_raw, variantxgene_effect)
        g_t = (gathered.reshape(L_PH // 8, B // 128, 8, 128)
               .transpose(0, 2, 1, 3).reshape(L_PH, B))

        in_specs = [
            pl.BlockSpec((L_PH, N_EMB, B_BLK),
                         functools.partial(lambda k, i: (k, 0, i), k)),
            pl.BlockSpec((L_PH, B_BLK), lambda i: (0, i)),
            pl.BlockSpec(memory_space=pltpu.SMEM),
            pl.BlockSpec(memory_space=pltpu.SMEM),
        ]
        args = [x_t, g_t, W.reshape(N_EMB), b]
        aliases = {}
        if acc is not None:
            in_specs.append(pl.BlockSpec(memory_space=pl.ANY))
            args.append(acc)
            aliases = {4: 0}
        acc = pl.pallas_call(
            _tc_body,
            grid=(GRID,),
            in_specs=in_specs,
            out_specs=pl.BlockSpec(
                (L_PH, B_BLK),
                functools.partial(lambda k, i: (k, i), k)),
            out_shape=jax.ShapeDtypeStruct((L, B), jnp.float32),
            input_output_aliases=aliases,
        )(*args)

    return acc.T


# speedup vs baseline: 1.0430x; 1.0430x over previous
import functools

import jax
import jax.numpy as jnp
from jax import lax
from jax.experimental import pallas as pl
from jax.experimental.pallas import tpu as pltpu
from jax.experimental.pallas import tpu_sc as plsc

N_EMB = 16
N_VXG = 1000000
B = 16384
L = 200
N = B * L

PHASE_ROWS = (8, 24, 40, 56, 72)
PHASE_L0 = (0, 8, 32, 72, 128)

NC = 2
NS = 16
NW = NC * NS
MAX_CHUNK = 14336


def _n_chunks(per_w):
    n = 1
    while per_w // n > MAX_CHUNK or per_w % n:
        n += 1
    return n


def _sc_gather(base_el, per_w, chunk, n_chunks,
               ixs_hbm, eff_hbm, out_hbm,
               idx_v0, idx_v1, val_v0, val_v1, eff_sh,
               sem_i0, sem_i1, sem_g, sem_o0, sem_o1):
    sid = lax.axis_index("s")
    wid = sid * NC + lax.axis_index("c")
    base = base_el + wid * per_w
    obase = wid * per_w

    @pl.when(sid == 0)
    def _():
        pltpu.sync_copy(eff_hbm, eff_sh)

    idx = [idx_v0, idx_v1]
    val = [val_v0, val_v1]
    sem_i = [sem_i0, sem_i1]
    sem_o = [sem_o0, sem_o1]

    pltpu.async_copy(ixs_hbm.at[pl.ds(base, chunk)], idx_v0, sem_i0)
    plsc.subcore_barrier()

    for ci in range(n_chunks):
        cur = ci % 2
        nxt = 1 - cur
        if ci + 1 < n_chunks:
            pltpu.async_copy(
                ixs_hbm.at[pl.ds(base + (ci + 1) * chunk, chunk)],
                idx[nxt], sem_i[nxt])
        pltpu.make_async_copy(
            ixs_hbm.at[pl.ds(base + ci * chunk, chunk)],
            idx[cur], sem_i[cur]).wait()
        if ci >= 2:
            pltpu.make_async_copy(
                val[cur], out_hbm.at[pl.ds(obase + (ci - 2) * chunk, chunk)],
                sem_o[cur]).wait()
        pltpu.async_copy(eff_sh.at[idx[cur]], val[cur], sem_g).wait()
        pltpu.async_copy(
            val[cur], out_hbm.at[pl.ds(obase + ci * chunk, chunk)],
            sem_o[cur])

    for ci in range(max(n_chunks - 2, 0), n_chunks):
        cur = ci % 2
        pltpu.make_async_copy(
            val[cur], out_hbm.at[pl.ds(obase + ci * chunk, chunk)],
            sem_o[cur]).wait()


def _tc_body(x_ref, g_ref, w_ref, b_ref, *rest):
    o_ref = rest[-1]
    acc = x_ref[:, 0, :] * w_ref[0]
    for k in range(1, N_EMB):
        acc = acc + x_ref[:, k, :] * w_ref[k]
    o_ref[...] = jnp.exp(acc + b_ref[0]) * g_ref[...]


def kernel(variantxgene_embedding, variantxgene_ixs, W, b, variantxgene_effect):
    ixs_raw = (variantxgene_ixs.T.reshape(L // 8, 8, B // 128, 128)
               .transpose(0, 2, 1, 3).reshape(N).astype(jnp.int32))
    x_t = jnp.transpose(variantxgene_embedding, (1, 2, 0))

    mesh = plsc.VectorSubcoreMesh(core_axis_name="c", subcore_axis_name="s")
    acc = None
    for ph in range(len(PHASE_ROWS)):
        l_ph = PHASE_ROWS[ph]
        l0 = PHASE_L0[ph]
        n_ph = l_ph * B
        per_w = n_ph // NW
        n_chunks = _n_chunks(per_w)
        chunk = per_w // n_chunks

        gather_run = functools.partial(
            pl.kernel,
            mesh=mesh,
            out_type=jax.ShapeDtypeStruct((n_ph,), jnp.float32),
            scratch_types=[
                pltpu.VMEM((chunk,), jnp.int32),
                pltpu.VMEM((chunk,), jnp.int32),
                pltpu.VMEM((chunk,), jnp.float32),
                pltpu.VMEM((chunk,), jnp.float32),
                pltpu.VMEM_SHARED((N_VXG,), jnp.float32),
                pltpu.SemaphoreType.DMA,
                pltpu.SemaphoreType.DMA,
                pltpu.SemaphoreType.DMA,
                pltpu.SemaphoreType.DMA,
                pltpu.SemaphoreType.DMA,
            ],
        )(functools.partial(_sc_gather, l0 * B, per_w, chunk, n_chunks))
        gathered = gather_run(ixs_raw, variantxgene_effect)
        g_t = (gathered.reshape(l_ph // 8, B // 128, 8, 128)
               .transpose(0, 2, 1, 3).reshape(l_ph, B))

        grid = l_ph // 8
        in_specs = [
            pl.BlockSpec((8, N_EMB, B),
                         functools.partial(lambda t0, i: (i + t0, 0, 0), l0 // 8)),
            pl.BlockSpec((8, B), lambda i: (i, 0)),
            pl.BlockSpec(memory_space=pltpu.SMEM),
            pl.BlockSpec(memory_space=pltpu.SMEM),
        ]
        args = [x_t, g_t, W.reshape(N_EMB), b]
        aliases = {}
        if acc is not None:
            in_specs.append(pl.BlockSpec(memory_space=pl.ANY))
            args.append(acc)
            aliases = {4: 0}
        acc = pl.pallas_call(
            _tc_body,
            grid=(grid,),
            in_specs=in_specs,
            out_specs=pl.BlockSpec(
                (8, B),
                functools.partial(lambda t0, i: (i + t0, 0), l0 // 8)),
            out_shape=jax.ShapeDtypeStruct((L, B), jnp.float32),
            input_output_aliases=aliases,
        )(*args)

    return acc.T


# speedup vs baseline: 1.0987x; 1.0534x over previous
import functools

import jax
import jax.numpy as jnp
from jax import lax
from jax.experimental import pallas as pl
from jax.experimental.pallas import tpu as pltpu
from jax.experimental.pallas import tpu_sc as plsc

N_EMB = 16
N_VXG = 1000000
B = 16384
L = 200
N = B * L

K_PHASES = 5
L_PH = L // K_PHASES
N_PH = L_PH * B

NC = 2
NS = 16
NW = NC * NS
N_PER_W = N_PH // NW
CHUNK = 10240
N_CHUNKS = N_PER_W // CHUNK

B_BLK = 4096
GRID = B // B_BLK


def _sc_gather(phase, ixs_hbm, eff_hbm, out_hbm,
               idx_v0, idx_v1, val_v0, val_v1, eff_sh,
               sem_i0, sem_i1, sem_g, sem_o0, sem_o1):
    sid = lax.axis_index("s")
    wid = sid * NC + lax.axis_index("c")
    base = phase * N_PH + wid * N_PER_W
    obase = wid * N_PER_W

    @pl.when(sid == 0)
    def _():
        pltpu.sync_copy(eff_hbm, eff_sh)

    idx = [idx_v0, idx_v1]
    val = [val_v0, val_v1]
    sem_i = [sem_i0, sem_i1]
    sem_o = [sem_o0, sem_o1]

    pltpu.async_copy(ixs_hbm.at[pl.ds(base, CHUNK)], idx_v0, sem_i0)
    plsc.subcore_barrier()

    for ci in range(N_CHUNKS):
        cur = ci % 2
        nxt = 1 - cur
        if ci + 1 < N_CHUNKS:
            pltpu.async_copy(
                ixs_hbm.at[pl.ds(base + (ci + 1) * CHUNK, CHUNK)],
                idx[nxt], sem_i[nxt])
        pltpu.make_async_copy(
            ixs_hbm.at[pl.ds(base + ci * CHUNK, CHUNK)],
            idx[cur], sem_i[cur]).wait()
        if ci >= 2:
            pltpu.make_async_copy(
                val[cur], out_hbm.at[pl.ds(obase + (ci - 2) * CHUNK, CHUNK)],
                sem_o[cur]).wait()
        pltpu.async_copy(eff_sh.at[idx[cur]], val[cur], sem_g).wait()
        pltpu.async_copy(
            val[cur], out_hbm.at[pl.ds(obase + ci * CHUNK, CHUNK)],
            sem_o[cur])

    for ci in range(max(N_CHUNKS - 2, 0), N_CHUNKS):
        cur = ci % 2
        pltpu.make_async_copy(
            val[cur], out_hbm.at[pl.ds(obase + ci * CHUNK, CHUNK)],
            sem_o[cur]).wait()


def _tc_body(x_ref, g_ref, w_ref, b_ref, *rest):
    o_ref = rest[-1]
    acc = x_ref[:, 0, :] * w_ref[0]
    for k in range(1, N_EMB):
        acc = acc + x_ref[:, k, :] * w_ref[k]
    o_ref[...] = jnp.exp(acc + b_ref[0]) * g_ref[...]


def kernel(variantxgene_embedding, variantxgene_ixs, W, b, variantxgene_effect):
    ixs_raw = (variantxgene_ixs.T.reshape(L // 8, 8, B // 128, 128)
               .transpose(0, 2, 1, 3).reshape(N).astype(jnp.int32))
    x_t = jnp.transpose(variantxgene_embedding, (1, 2, 0))

    mesh = plsc.VectorSubcoreMesh(core_axis_name="c", subcore_axis_name="s")
    acc = None
    for k in range(K_PHASES):
        gather_run = functools.partial(
            pl.kernel,
            mesh=mesh,
            out_type=jax.ShapeDtypeStruct((N_PH,), jnp.float32),
            scratch_types=[
                pltpu.VMEM((CHUNK,), jnp.int32),
                pltpu.VMEM((CHUNK,), jnp.int32),
                pltpu.VMEM((CHUNK,), jnp.float32),
                pltpu.VMEM((CHUNK,), jnp.float32),
                pltpu.VMEM_SHARED((N_VXG,), jnp.float32),
                pltpu.SemaphoreType.DMA,
                pltpu.SemaphoreType.DMA,
                pltpu.SemaphoreType.DMA,
                pltpu.SemaphoreType.DMA,
                pltpu.SemaphoreType.DMA,
            ],
        )(functools.partial(_sc_gather, k))
        gathered = gather_run(ixs_raw, variantxgene_effect)
        g_t = (gathered.reshape(L_PH // 8, B // 128, 8, 128)
               .transpose(0, 2, 1, 3).reshape(L_PH, B))

        in_specs = [
            pl.BlockSpec((L_PH, N_EMB, B_BLK),
                         functools.partial(lambda k, i: (k, 0, i), k)),
            pl.BlockSpec((L_PH, B_BLK), lambda i: (0, i)),
            pl.BlockSpec(memory_space=pltpu.SMEM),
            pl.BlockSpec(memory_space=pltpu.SMEM),
        ]
        args = [x_t, g_t, W.reshape(N_EMB), b]
        aliases = {}
        if acc is not None:
            in_specs.append(pl.BlockSpec(memory_space=pl.ANY))
            args.append(acc)
            aliases = {4: 0}
        acc = pl.pallas_call(
            _tc_body,
            grid=(GRID,),
            in_specs=in_specs,
            out_specs=pl.BlockSpec(
                (L_PH, B_BLK),
                functools.partial(lambda k, i: (k, i), k)),
            out_shape=jax.ShapeDtypeStruct((L, B), jnp.float32),
            input_output_aliases=aliases,
        )(*args)

    return acc.T


# speedup vs baseline: 1.1062x; 1.0069x over previous
import functools

import jax
import jax.numpy as jnp
from jax import lax
from jax.experimental import pallas as pl
from jax.experimental.pallas import tpu as pltpu
from jax.experimental.pallas import tpu_sc as plsc

N_EMB = 16
N_VXG = 1000000
B = 16384
L = 200
N = B * L

K_PHASES = 5
L_PH = L // K_PHASES
N_PH = L_PH * B

NC = 2
NS = 16
NW = NC * NS
N_PER_W = N_PH // NW
CHUNK = 10240
N_CHUNKS = N_PER_W // CHUNK

B_BLK = 2048
GRID = B // B_BLK


def _sc_gather(phase, ixs_hbm, eff_hbm, out_hbm,
               idx_v0, idx_v1, val_v0, val_v1, eff_sh,
               sem_i0, sem_i1, sem_g, sem_o0, sem_o1):
    sid = lax.axis_index("s")
    wid = sid * NC + lax.axis_index("c")
    base = phase * N_PH + wid * N_PER_W
    obase = wid * N_PER_W

    @pl.when(sid == 0)
    def _():
        pltpu.sync_copy(eff_hbm, eff_sh)

    idx = [idx_v0, idx_v1]
    val = [val_v0, val_v1]
    sem_i = [sem_i0, sem_i1]
    sem_o = [sem_o0, sem_o1]

    pltpu.async_copy(ixs_hbm.at[pl.ds(base, CHUNK)], idx_v0, sem_i0)
    plsc.subcore_barrier()

    for ci in range(N_CHUNKS):
        cur = ci % 2
        nxt = 1 - cur
        if ci + 1 < N_CHUNKS:
            pltpu.async_copy(
                ixs_hbm.at[pl.ds(base + (ci + 1) * CHUNK, CHUNK)],
                idx[nxt], sem_i[nxt])
        pltpu.make_async_copy(
            ixs_hbm.at[pl.ds(base + ci * CHUNK, CHUNK)],
            idx[cur], sem_i[cur]).wait()
        if ci >= 2:
            pltpu.make_async_copy(
                val[cur], out_hbm.at[pl.ds(obase + (ci - 2) * CHUNK, CHUNK)],
                sem_o[cur]).wait()
        pltpu.async_copy(eff_sh.at[idx[cur]], val[cur], sem_g).wait()
        pltpu.async_copy(
            val[cur], out_hbm.at[pl.ds(obase + ci * CHUNK, CHUNK)],
            sem_o[cur])

    for ci in range(max(N_CHUNKS - 2, 0), N_CHUNKS):
        cur = ci % 2
        pltpu.make_async_copy(
            val[cur], out_hbm.at[pl.ds(obase + ci * CHUNK, CHUNK)],
            sem_o[cur]).wait()


def _tc_body(x_ref, g_ref, w_ref, b_ref, *rest):
    o_ref = rest[-1]
    acc = x_ref[:, 0, :] * w_ref[0]
    for k in range(1, N_EMB):
        acc = acc + x_ref[:, k, :] * w_ref[k]
    o_ref[...] = jnp.exp(acc + b_ref[0]) * g_ref[...]


def kernel(variantxgene_embedding, variantxgene_ixs, W, b, variantxgene_effect):
    ixs_raw = (variantxgene_ixs.T.reshape(L // 8, 8, B // 128, 128)
               .transpose(0, 2, 1, 3).reshape(N).astype(jnp.int32))
    x_t = jnp.transpose(variantxgene_embedding, (1, 2, 0))

    mesh = plsc.VectorSubcoreMesh(core_axis_name="c", subcore_axis_name="s")
    acc = None
    for k in range(K_PHASES):
        gather_run = functools.partial(
            pl.kernel,
            mesh=mesh,
            out_type=jax.ShapeDtypeStruct((N_PH,), jnp.float32),
            scratch_types=[
                pltpu.VMEM((CHUNK,), jnp.int32),
                pltpu.VMEM((CHUNK,), jnp.int32),
                pltpu.VMEM((CHUNK,), jnp.float32),
                pltpu.VMEM((CHUNK,), jnp.float32),
                pltpu.VMEM_SHARED((N_VXG,), jnp.float32),
                pltpu.SemaphoreType.DMA,
                pltpu.SemaphoreType.DMA,
                pltpu.SemaphoreType.DMA,
                pltpu.SemaphoreType.DMA,
                pltpu.SemaphoreType.DMA,
            ],
        )(functools.partial(_sc_gather, k))
        gathered = gather_run(ixs_raw, variantxgene_effect)
        g_t = (gathered.reshape(L_PH // 8, B // 128, 8, 128)
               .transpose(0, 2, 1, 3).reshape(L_PH, B))

        in_specs = [
            pl.BlockSpec((L_PH, N_EMB, B_BLK),
                         functools.partial(lambda k, i: (k, 0, i), k)),
            pl.BlockSpec((L_PH, B_BLK), lambda i: (0, i)),
            pl.BlockSpec(memory_space=pltpu.SMEM),
            pl.BlockSpec(memory_space=pltpu.SMEM),
        ]
        args = [x_t, g_t, W.reshape(N_EMB), b]
        aliases = {}
        if acc is not None:
            in_specs.append(pl.BlockSpec(memory_space=pl.ANY))
            args.append(acc)
            aliases = {4: 0}
        acc = pl.pallas_call(
            _tc_body,
            grid=(GRID,),
            in_specs=in_specs,
            out_specs=pl.BlockSpec(
                (L_PH, B_BLK),
                functools.partial(lambda k, i: (k, i), k)),
            out_shape=jax.ShapeDtypeStruct((L, B), jnp.float32),
            input_output_aliases=aliases,
        )(*args)

    return acc.T
